# baseline (speedup 1.0000x reference)
import jax
import jax.numpy as jnp
from jax import lax
from jax.experimental import pallas as pl
from jax.experimental.pallas import tpu as pltpu
from jax.experimental.pallas import tpu_sc as plsc

_D = 64
_BLK = 128
_NB_TOTAL = 6400
_K = 4
_NC = 2
_NS = 16
_NW = _NC * _NS
_NB_PER_W = _NB_TOTAL // _NW
_G = _NB_PER_W // _K


def _body(idx_hbm, table_hbm, out_hbm, idx_v, rows_v, gsem):
    c = lax.axis_index("c")
    s = lax.axis_index("s")
    wid = s * _NC + c
    blk_base = wid * _NB_PER_W

    def chunk(g, carry):
        blk0 = blk_base + g * _K
        pltpu.sync_copy(idx_hbm.at[pl.ds(blk0, _K)], idx_v)
        for k in range(_K):
            pltpu.async_copy(table_hbm.at[idx_v.at[k]], rows_v.at[k], gsem)
        for k in range(_K):
            pltpu.make_async_copy(table_hbm.at[idx_v.at[k]], rows_v.at[k],
                                  gsem).wait()

        def srow(r, c2):
            for k in range(_K):
                for cc in range(_D // 16):
                    sl = pl.ds(cc * 16, 16)
                    rows_v[k, r, sl] = rows_v[k, r, sl] * 8.0
            return c2

        lax.fori_loop(0, _BLK, srow, 0)
        pltpu.sync_copy(rows_v, out_hbm.at[pl.ds(blk0, _K)])
        return carry

    lax.fori_loop(0, _G, chunk, 0)


def kernel(x, embedding):
    b, l = x.shape
    idx = x.reshape(_NB_TOTAL, _BLK).astype(jnp.int32)
    mesh = plsc.VectorSubcoreMesh(core_axis_name="c", subcore_axis_name="s")
    out = pl.kernel(
        _body,
        out_type=jax.ShapeDtypeStruct((_NB_TOTAL, _BLK, _D), jnp.float32),
        mesh=mesh,
        scratch_types=[
            pltpu.VMEM((_K, _BLK), jnp.int32),
            pltpu.VMEM((_K, _BLK, _D), jnp.float32),
            pltpu.SemaphoreType.DMA,
        ],
        compiler_params=pltpu.CompilerParams(use_tc_tiling_on_sc=False),
    )(idx, embedding)
    return out.reshape(b, l, _D)


# speedup vs baseline: 1.0905x; 1.0905x over previous
import jax
import jax.numpy as jnp
from jax import lax
from jax.experimental import pallas as pl
from jax.experimental.pallas import tpu as pltpu
from jax.experimental.pallas import tpu_sc as plsc

_D = 64
_BLK = 128
_NB_TOTAL = 6400
_K = 4
_NC = 2
_NS = 16
_NW = _NC * _NS
_NB_PER_W = _NB_TOTAL // _NW
_G = _NB_PER_W // _K
_NBUF = 3
_ROUNDS = (_G + _NBUF - 1) // _NBUF


def _body(idx_hbm, table_hbm, out_hbm, idx_v, rows_v, gsem, osem):
    c = lax.axis_index("c")
    s = lax.axis_index("s")
    wid = s * _NC + c
    blk_base = wid * _NB_PER_W

    pltpu.sync_copy(idx_hbm.at[pl.ds(blk_base, _NB_PER_W)], idx_v)

    def fire_gather(g, slot):
        for k in range(_K):
            pltpu.async_copy(table_hbm.at[idx_v.at[g * _K + k]],
                             rows_v.at[slot, k], gsem.at[slot])

    def drain_gather(g, slot):
        for k in range(_K):
            pltpu.make_async_copy(table_hbm.at[idx_v.at[g * _K + k]],
                                  rows_v.at[slot, k], gsem.at[slot]).wait()

    def fire_out(g, slot):
        pltpu.async_copy(rows_v.at[slot],
                         out_hbm.at[pl.ds(blk_base + g * _K, _K)],
                         osem.at[slot])

    def drain_out(g, slot):
        pltpu.make_async_copy(rows_v.at[slot],
                              out_hbm.at[pl.ds(blk_base + g * _K, _K)],
                              osem.at[slot]).wait()

    def scale(slot):
        def srow(r, c2):
            for k in range(_K):
                for cc in range(_D // 16):
                    sl = pl.ds(cc * 16, 16)
                    rows_v[slot, k, r, sl] = rows_v[slot, k, r, sl] * 8.0
            return c2

        lax.fori_loop(0, _BLK, srow, 0)

    fire_gather(0, 0)
    fire_gather(1, 1)

    def round_(t, carry):
        for b in range(_NBUF):
            g = t * _NBUF + b

            @pl.when(g < _G)
            def _():
                drain_gather(g, b)
                scale(b)
                fire_out(g, b)

                @pl.when(g > 0)
                def _():
                    drain_out(g - 1, (b - 1) % _NBUF)

                @pl.when(g + 2 < _G)
                def _():
                    fire_gather(g + 2, (b + 2) % _NBUF)

        return carry

    lax.fori_loop(0, _ROUNDS, round_, 0)
    drain_out(_G - 1, (_G - 1) % _NBUF)


def kernel(x, embedding):
    b, l = x.shape
    idx = x.reshape(_NB_TOTAL, _BLK).astype(jnp.int32)
    mesh = plsc.VectorSubcoreMesh(core_axis_name="c", subcore_axis_name="s")
    out = pl.kernel(
        _body,
        out_type=jax.ShapeDtypeStruct((_NB_TOTAL, _BLK, _D), jnp.float32),
        mesh=mesh,
        scratch_types=[
            pltpu.VMEM((_NB_PER_W, _BLK), jnp.int32),
            pltpu.VMEM((_NBUF, _K, _BLK, _D), jnp.float32),
            pltpu.SemaphoreType.DMA((_NBUF,)),
            pltpu.SemaphoreType.DMA((_NBUF,)),
        ],
        compiler_params=pltpu.CompilerParams(use_tc_tiling_on_sc=False),
    )(idx, embedding)
    return out.reshape(b, l, _D)
